# baseline (speedup 1.0000x reference)
import jax
import jax.numpy as jnp
from jax import lax
from jax.experimental import pallas as pl
from jax.experimental.pallas import tpu as pltpu
from jax.experimental.pallas import tpu_sc as plsc

F = 26
V = 100000
D = 32
B = 16384

NC = 2
NS = 16
NW = NC * NS
BW = B // NW
CHUNK = 128
NCH = BW // CHUNK
LANES = 16


def _body(x_hbm, tab_hbm, out_hbm, xv, idxv, acc, sem):
    wid = lax.axis_index("s") * NC + lax.axis_index("c")
    base = wid * BW

    pltpu.sync_copy(x_hbm.at[pl.ds(base * F, BW * F)], xv)

    lane = lax.iota(jnp.int32, LANES)
    jper = CHUNK // LANES

    def build_f(f, _):
        off = f * V

        def build_j(j, _):
            flat = (j * LANES + lane) * F + f
            vals = plsc.load_gather(xv, [flat]) + off
            r = f * NCH + j // jper
            idxv[r, pl.ds((j % jper) * LANES, LANES)] = vals
            return 0

        lax.fori_loop(0, BW // LANES, build_j, 0, unroll=4)
        return 0

    lax.fori_loop(0, F, build_f, 0)

    for c in range(NCH):
        accc = acc.at[pl.ds(c * CHUNK, CHUNK)]
        pltpu.async_copy(tab_hbm.at[idxv.at[c]], accc, sem).wait()

        def gather_f(f, _):
            pltpu.async_copy(
                tab_hbm.at[idxv.at[f * NCH + c]], accc, sem, add=True
            ).wait()
            return 0

        lax.fori_loop(1, F, gather_f, 0)

    pltpu.sync_copy(acc, out_hbm.at[pl.ds(base, BW)])


def kernel(x, tables):
    flat_tab = tables.reshape(F * V, D)
    flat_x = x.reshape(B * F)
    run = pl.kernel(
        _body,
        out_type=jax.ShapeDtypeStruct((B, D), jnp.float32),
        compiler_params=pltpu.CompilerParams(
            needs_layout_passes=False, use_tc_tiling_on_sc=False,
        ),
        mesh=plsc.VectorSubcoreMesh(
            core_axis_name="c", subcore_axis_name="s",
            num_cores=NC, num_subcores=NS,
        ),
        scratch_types=[
            pltpu.VMEM((BW * F,), jnp.int32),
            pltpu.VMEM((F * NCH, CHUNK), jnp.int32),
            pltpu.VMEM((BW, D), jnp.float32),
            pltpu.SemaphoreType.DMA,
        ],
    )
    return run(flat_x, flat_tab)


# speedup vs baseline: 1.0511x; 1.0511x over previous
import jax
import jax.numpy as jnp
from jax import lax
from jax.experimental import pallas as pl
from jax.experimental.pallas import tpu as pltpu
from jax.experimental.pallas import tpu_sc as plsc

F = 26
V = 100000
D = 32
B = 16384

NC = 2
NS = 16
NW = NC * NS
BW = B // NW
CHUNK = 128
NCH = BW // CHUNK
LANES = 16


def _body(x_hbm, tab_hbm, out_hbm, xv, idxv, acc, sem):
    wid = lax.axis_index("s") * NC + lax.axis_index("c")
    base = wid * BW

    pltpu.sync_copy(x_hbm.at[pl.ds(base * F, BW * F)], xv)

    lane = lax.iota(jnp.int32, LANES)
    jper = CHUNK // LANES

    def build_f(f, _):
        off = f * V

        def build_j(j, _):
            flat = (j * LANES + lane) * F + f
            vals = plsc.load_gather(xv, [flat]) + off
            r = f * NCH + j // jper
            idxv[r, pl.ds((j % jper) * LANES, LANES)] = vals
            return 0

        lax.fori_loop(0, BW // LANES, build_j, 0, unroll=4)
        return 0

    lax.fori_loop(0, F, build_f, 0)

    descs = [
        pltpu.async_copy(
            tab_hbm.at[idxv.at[c]], acc.at[pl.ds(c * CHUNK, CHUNK)], sem
        )
        for c in range(NCH)
    ]
    for d in descs:
        d.wait()

    for c in range(NCH):
        accc = acc.at[pl.ds(c * CHUNK, CHUNK)]

        def fire_f(f, _):
            pltpu.async_copy(
                tab_hbm.at[idxv.at[f * NCH + c]], accc, sem, add=True
            )
            return 0

        lax.fori_loop(1, F, fire_f, 0)

    def drain(i, _):
        pltpu.make_async_copy(
            tab_hbm.at[idxv.at[0]], acc.at[pl.ds(0, CHUNK)], sem
        ).wait()
        return 0

    lax.fori_loop(0, (F - 1) * NCH, drain, 0)

    pltpu.sync_copy(acc, out_hbm.at[pl.ds(base, BW)])


def kernel(x, tables):
    flat_tab = tables.reshape(F * V, D)
    flat_x = x.reshape(B * F)
    run = pl.kernel(
        _body,
        out_type=jax.ShapeDtypeStruct((B, D), jnp.float32),
        compiler_params=pltpu.CompilerParams(
            needs_layout_passes=False, use_tc_tiling_on_sc=False,
        ),
        mesh=plsc.VectorSubcoreMesh(
            core_axis_name="c", subcore_axis_name="s",
            num_cores=NC, num_subcores=NS,
        ),
        scratch_types=[
            pltpu.VMEM((BW * F,), jnp.int32),
            pltpu.VMEM((F * NCH, CHUNK), jnp.int32),
            pltpu.VMEM((BW, D), jnp.float32),
            pltpu.SemaphoreType.DMA,
        ],
    )
    return run(flat_x, flat_tab)
